# Initial kernel scaffold; baseline (speedup 1.0000x reference)
import jax
import jax.numpy as jnp
from jax.experimental import pallas as pl


def kernel(y_pred, y_true):
    raise NotImplementedError("write your pallas kernel here")


# speedup vs baseline: 21.5986x; 21.5986x over previous
import functools

import jax
import jax.numpy as jnp
from jax import lax
from jax.experimental import pallas as pl
from jax.experimental.pallas import tpu as pltpu
from jax.experimental.pallas import tpu_sc as plsc

_N = 16 * 512 * 512
_B = 16384
_NC, _NS, _L = 2, 16, 16
_NW = _NC * _NS
_PER_W = _N // _NW
_CH = 2048
_N_CH = _PER_W // _CH
_UNROLL = 4


def _mm_body(p_ref, l_ref, mn_ref, mx_ref):
    i = pl.program_id(0)
    lf = l_ref[...].astype(jnp.float32)
    e = 1.0 - p_ref[...] * (2.0 * lf - 1.0)
    mn = jnp.min(e)
    mx = jnp.max(e)

    @pl.when(i == 0)
    def _():
        mn_ref[...] = jnp.full((8, 128), mn, jnp.float32)
        mx_ref[...] = jnp.full((8, 128), mx, jnp.float32)

    @pl.when(i != 0)
    def _():
        mn_ref[...] = jnp.minimum(mn_ref[...], mn)
        mx_ref[...] = jnp.maximum(mx_ref[...], mx)


def _minmax(pred2d, lab2d):
    rows = pred2d.shape[0]
    blk = 512
    grid = rows // blk
    return pl.pallas_call(
        _mm_body,
        grid=(grid,),
        in_specs=[
            pl.BlockSpec((blk, 1024), lambda i: (i, 0)),
            pl.BlockSpec((blk, 1024), lambda i: (i, 0)),
        ],
        out_specs=[
            pl.BlockSpec((8, 128), lambda i: (0, 0)),
            pl.BlockSpec((8, 128), lambda i: (0, 0)),
        ],
        out_shape=[
            jax.ShapeDtypeStruct((8, 128), jnp.float32),
            jax.ShapeDtypeStruct((8, 128), jnp.float32),
        ],
    )(pred2d, lab2d)


def _hist_body(pred_hbm, lab_hbm, mn_hbm, mx_hbm, h_out, s_out,
               pbuf, lbuf, mnb, mxb, hvm, svm):
    wid = lax.axis_index("s") * _NC + lax.axis_index("c")

    def zero_body(j, carry):
        z = jnp.zeros((_L,), jnp.float32)
        hvm[pl.ds(j * _L, _L)] = z
        svm[pl.ds(j * _L, _L)] = z
        return carry

    lax.fori_loop(0, (2 * _B) // _L, zero_body, 0)

    pltpu.sync_copy(mn_hbm, mnb)
    pltpu.sync_copy(mx_hbm, mxb)
    mn = mnb[...]
    mx = mxb[...]
    scale = (_B - 1.0) / jnp.maximum(mx - mn, 1e-30)
    ones = jnp.full((_L,), 1.0, jnp.float32)
    top = jnp.full((_L,), _B - 1.0, jnp.float32)
    zero = jnp.zeros((_L,), jnp.float32)

    def chunk_body(c, carry):
        base = wid * _PER_W + c * _CH
        pltpu.sync_copy(pred_hbm.at[pl.ds(base, _CH)], pbuf)
        pltpu.sync_copy(lab_hbm.at[pl.ds(base, _CH)], lbuf)

        def vec_body(j, inner):
            for u in range(_UNROLL):
                off = (j * _UNROLL + u) * _L
                p = pbuf[pl.ds(off, _L)]
                lab = lbuf[pl.ds(off, _L)]
                labf = lab.astype(jnp.float32)
                e = 1.0 - p * (2.0 * labf - 1.0)
                t = (mx - e) * scale
                t = jnp.minimum(jnp.maximum(t, zero), top)
                idx = t.astype(jnp.int32) + lab * _B
                plsc.addupdate_scatter(hvm, [idx], ones)
                plsc.addupdate_scatter(svm, [idx], e)
            return inner

        lax.fori_loop(0, _CH // (_L * _UNROLL), vec_body, 0)
        return carry

    lax.fori_loop(0, _N_CH, chunk_body, 0)
    pltpu.sync_copy(hvm, h_out.at[wid])
    pltpu.sync_copy(svm, s_out.at[wid])


def _histogram(pred_flat, lab_flat, mnv, mxv):
    mesh = plsc.VectorSubcoreMesh(core_axis_name="c", subcore_axis_name="s")
    return pl.kernel(
        _hist_body,
        mesh=mesh,
        compiler_params=pltpu.CompilerParams(needs_layout_passes=False),
        out_type=[
            jax.ShapeDtypeStruct((_NW, 2 * _B), jnp.float32),
            jax.ShapeDtypeStruct((_NW, 2 * _B), jnp.float32),
        ],
        scratch_types=[
            pltpu.VMEM((_CH,), jnp.float32),
            pltpu.VMEM((_CH,), jnp.int32),
            pltpu.VMEM((_L,), jnp.float32),
            pltpu.VMEM((_L,), jnp.float32),
            pltpu.VMEM((2 * _B,), jnp.float32),
            pltpu.VMEM((2 * _B,), jnp.float32),
        ],
    )(pred_flat, lab_flat, mnv, mxv)


def _excl_prefix(x):
    inc = x
    for k in (1, 2, 4, 8, 16, 32, 64):
        shifted = jnp.concatenate(
            [jnp.zeros((128, k), jnp.float32), inc[:, : 128 - k]], axis=1)
        inc = inc + shifted
    rowtot = jnp.broadcast_to(inc[:, 127:128], (128, 128))
    cumr = rowtot
    for k in (1, 2, 4, 8, 16, 32, 64):
        shifted = jnp.concatenate(
            [jnp.zeros((k, 128), jnp.float32), cumr[: 128 - k, :]], axis=0)
        cumr = cumr + shifted
    return inc - x + (cumr - rowtot)


def _f(e):
    return jnp.where(e > 0.0, e + 1.0, jnp.exp(e))


def _fin_body(h_ref, s_ref, mx_ref, o_ref):
    Hn = jnp.sum(h_ref[:, 0], axis=0)
    Hp = jnp.sum(h_ref[:, 1], axis=0)
    Sn = jnp.sum(s_ref[:, 0], axis=0)
    Sp = jnp.sum(s_ref[:, 1], axis=0)
    G = jnp.sum(Hp)
    Ppos = _excl_prefix(Hp)
    Pneg = _excl_prefix(Hn)
    u0 = G + Pneg
    fp = _f(Sp / jnp.maximum(Hp, 1.0))
    fn = _f(Sn / jnp.maximum(Hn, 1.0))
    pos_t = Hp * fp / jnp.maximum(u0, 1.0)
    I = G - Ppos - Hp
    neg_t = Hn * fn * I / jnp.maximum(u0 * (u0 + Hn), 1.0)
    loss = jnp.sum(pos_t) + jnp.sum(neg_t)
    mxs = jnp.max(mx_ref[...])
    loss = jnp.where(G == 0.0, _f(mxs), loss)
    o_ref[0, 0] = loss


def _finish(h4, s4, mxv):
    return pl.pallas_call(
        _fin_body,
        in_specs=[
            pl.BlockSpec(memory_space=pltpu.VMEM),
            pl.BlockSpec(memory_space=pltpu.VMEM),
            pl.BlockSpec(memory_space=pltpu.VMEM),
        ],
        out_specs=pl.BlockSpec(memory_space=pltpu.SMEM),
        out_shape=jax.ShapeDtypeStruct((1, 1), jnp.float32),
    )(h4, s4, mxv)


def kernel(y_pred, y_true):
    pred2d = y_pred.reshape(4096, 1024)
    lab2d = y_true.astype(jnp.int32).reshape(4096, 1024)
    mn8, mx8 = _minmax(pred2d, lab2d)
    mnv = mn8[0, :_L]
    mxv = mx8[0, :_L]
    h, s = _histogram(pred2d.reshape(-1), lab2d.reshape(-1), mnv, mxv)
    h4 = h.reshape(_NW, 2, 128, 128)
    s4 = s.reshape(_NW, 2, 128, 128)
    out = _finish(h4, s4, mx8)
    return out.reshape(())


# speedup vs baseline: 68.1018x; 3.1531x over previous
import functools

import jax
import jax.numpy as jnp
from jax import lax
from jax.experimental import pallas as pl
from jax.experimental.pallas import tpu as pltpu
from jax.experimental.pallas import tpu_sc as plsc

_N = 16 * 512 * 512
_B = 16384
_NC, _NS, _L = 2, 16, 16
_NW = _NC * _NS
_PER_W = _N // _NW
_CH = 4096
_N_CH = _PER_W // _CH
_GROUPS = _CH // _L


def _mm_body(p_ref, l_ref, mn_ref, mx_ref):
    i = pl.program_id(0)
    lf = l_ref[...].astype(jnp.float32)
    e = 1.0 - p_ref[...] * (2.0 * lf - 1.0)
    mn = jnp.min(e)
    mx = jnp.max(e)

    @pl.when(i == 0)
    def _():
        mn_ref[...] = jnp.full((8, 128), mn, jnp.float32)
        mx_ref[...] = jnp.full((8, 128), mx, jnp.float32)

    @pl.when(i != 0)
    def _():
        mn_ref[...] = jnp.minimum(mn_ref[...], mn)
        mx_ref[...] = jnp.maximum(mx_ref[...], mx)


def _minmax(pred2d, lab2d):
    rows = pred2d.shape[0]
    blk = 1024
    grid = rows // blk
    return pl.pallas_call(
        _mm_body,
        grid=(grid,),
        in_specs=[
            pl.BlockSpec((blk, 512), lambda i: (i, 0)),
            pl.BlockSpec((blk, 512), lambda i: (i, 0)),
        ],
        out_specs=[
            pl.BlockSpec((8, 128), lambda i: (0, 0)),
            pl.BlockSpec((8, 128), lambda i: (0, 0)),
        ],
        out_shape=[
            jax.ShapeDtypeStruct((8, 128), jnp.float32),
            jax.ShapeDtypeStruct((8, 128), jnp.float32),
        ],
    )(pred2d, lab2d)


def _hist_body(pred_hbm, lab_hbm, mn_hbm, mx_hbm, h_out, s_out,
               pb0, pb1, lb0, lb1, mnb, mxb, hvm, svm, sem0, sem1):
    wid = lax.axis_index("s") * _NC + lax.axis_index("c")
    base = wid * _PER_W

    @plsc.parallel_loop(0, (2 * _B) // _L, unroll=8)
    def _(j):
        z = jnp.zeros((_L,), jnp.float32)
        hvm[pl.ds(j * _L, _L)] = z
        svm[pl.ds(j * _L, _L)] = z

    pltpu.sync_copy(mn_hbm, mnb)
    pltpu.sync_copy(mx_hbm, mxb)
    mn = mnb[...]
    mx = mxb[...]
    scale = (_B - 1.0) / jnp.maximum(mx - mn, 1e-30)
    a_vec = (mx - 1.0) * scale
    two_scale = scale + scale
    ones = jnp.full((_L,), 1.0, jnp.float32)
    top = jnp.full((_L,), _B - 1.0, jnp.float32)
    zero = jnp.zeros((_L,), jnp.float32)

    def _start(c, pb, lb, sem):
        pltpu.async_copy(pred_hbm.at[pl.ds(base + c * _CH, _CH)], pb, sem)
        pltpu.async_copy(lab_hbm.at[pl.ds(base + c * _CH, _CH)], lb, sem)

    def _wait(c, pb, lb, sem):
        pltpu.make_async_copy(
            pred_hbm.at[pl.ds(base + c * _CH, _CH)], pb, sem).wait()
        pltpu.make_async_copy(
            lab_hbm.at[pl.ds(base + c * _CH, _CH)], lb, sem).wait()

    def _process(pb, lb):
        @plsc.parallel_loop(0, _GROUPS, unroll=8)
        def _(j):
            off = j * _L
            p = pb[pl.ds(off, _L)]
            lab = lb[pl.ds(off, _L)]
            lf = lab.astype(jnp.float32)
            t = a_vec + p * (lf * two_scale - scale)
            tc = jnp.minimum(jnp.maximum(t, zero), top)
            idx = tc.astype(jnp.int32) + lab * _B
            plsc.addupdate_scatter(hvm, [idx], ones)
            plsc.addupdate_scatter(svm, [idx], t)

    _start(0, pb0, lb0, sem0)

    def pair_body(cc, carry):
        c0 = 2 * cc
        c1 = c0 + 1
        _start(c1, pb1, lb1, sem1)
        _wait(c0, pb0, lb0, sem0)
        _process(pb0, lb0)
        cn = lax.rem(c0 + 2, _N_CH)
        _start(cn, pb0, lb0, sem0)
        _wait(c1, pb1, lb1, sem1)
        _process(pb1, lb1)
        return carry

    lax.fori_loop(0, _N_CH // 2, pair_body, 0)
    _wait(0, pb0, lb0, sem0)

    pltpu.sync_copy(hvm, h_out.at[wid])
    pltpu.sync_copy(svm, s_out.at[wid])


def _histogram(pred_flat, lab_flat, mnv, mxv):
    mesh = plsc.VectorSubcoreMesh(core_axis_name="c", subcore_axis_name="s")
    return pl.kernel(
        _hist_body,
        mesh=mesh,
        compiler_params=pltpu.CompilerParams(needs_layout_passes=False),
        out_type=[
            jax.ShapeDtypeStruct((_NW, 2 * _B), jnp.float32),
            jax.ShapeDtypeStruct((_NW, 2 * _B), jnp.float32),
        ],
        scratch_types=[
            pltpu.VMEM((_CH,), jnp.float32),
            pltpu.VMEM((_CH,), jnp.float32),
            pltpu.VMEM((_CH,), jnp.int32),
            pltpu.VMEM((_CH,), jnp.int32),
            pltpu.VMEM((_L,), jnp.float32),
            pltpu.VMEM((_L,), jnp.float32),
            pltpu.VMEM((2 * _B,), jnp.float32),
            pltpu.VMEM((2 * _B,), jnp.float32),
            pltpu.SemaphoreType.DMA,
            pltpu.SemaphoreType.DMA,
        ],
    )(pred_flat, lab_flat, mnv, mxv)


def _excl_prefix(x):
    inc = x
    for k in (1, 2, 4, 8, 16, 32, 64):
        shifted = jnp.concatenate(
            [jnp.zeros((128, k), jnp.float32), inc[:, : 128 - k]], axis=1)
        inc = inc + shifted
    rowtot = jnp.broadcast_to(inc[:, 127:128], (128, 128))
    cumr = rowtot
    for k in (1, 2, 4, 8, 16, 32, 64):
        shifted = jnp.concatenate(
            [jnp.zeros((k, 128), jnp.float32), cumr[: 128 - k, :]], axis=0)
        cumr = cumr + shifted
    return inc - x + (cumr - rowtot)


def _f(e):
    return jnp.where(e > 0.0, e + 1.0, jnp.exp(e))


def _fin_body(h_ref, s_ref, mn_ref, mx_ref, o_ref):
    Hn = jnp.sum(h_ref[:, 0], axis=0)
    Hp = jnp.sum(h_ref[:, 1], axis=0)
    Sn = jnp.sum(s_ref[:, 0], axis=0)
    Sp = jnp.sum(s_ref[:, 1], axis=0)
    G = jnp.sum(Hp)
    mns = jnp.max(mn_ref[...])
    mxs = jnp.max(mx_ref[...])
    scale = (_B - 1.0) / jnp.maximum(mxs - mns, 1e-30)
    ep = mxs - (Sp / jnp.maximum(Hp, 1.0)) / scale
    en = mxs - (Sn / jnp.maximum(Hn, 1.0)) / scale
    Ppos = _excl_prefix(Hp)
    Pneg = _excl_prefix(Hn)
    u0 = G + Pneg
    fp = _f(ep)
    fn = _f(en)
    pos_t = Hp * fp / jnp.maximum(u0, 1.0)
    I = G - Ppos - Hp
    neg_t = Hn * fn * I / jnp.maximum(u0 * (u0 + Hn), 1.0)
    loss = jnp.sum(pos_t) + jnp.sum(neg_t)
    loss = jnp.where(G == 0.0, _f(mxs), loss)
    o_ref[0, 0] = loss


def _finish(h4, s4, mn8, mx8):
    return pl.pallas_call(
        _fin_body,
        in_specs=[
            pl.BlockSpec(memory_space=pltpu.VMEM),
            pl.BlockSpec(memory_space=pltpu.VMEM),
            pl.BlockSpec(memory_space=pltpu.VMEM),
            pl.BlockSpec(memory_space=pltpu.VMEM),
        ],
        out_specs=pl.BlockSpec(memory_space=pltpu.SMEM),
        out_shape=jax.ShapeDtypeStruct((1, 1), jnp.float32),
    )(h4, s4, mn8, mx8)


def kernel(y_pred, y_true):
    lab = y_true.astype(jnp.int32)
    pred2d = y_pred.reshape(8192, 512)
    lab2d = lab.reshape(8192, 512)
    mn8, mx8 = _minmax(pred2d, lab2d)
    mnv = mn8[0, :_L]
    mxv = mx8[0, :_L]
    h, s = _histogram(y_pred.reshape(-1), lab.reshape(-1), mnv, mxv)
    h4 = h.reshape(_NW, 2, 128, 128)
    s4 = s.reshape(_NW, 2, 128, 128)
    out = _finish(h4, s4, mn8, mx8)
    return out.reshape(())


# speedup vs baseline: 90.6822x; 1.3316x over previous
import functools

import jax
import jax.numpy as jnp
from jax import lax
from jax.experimental import pallas as pl
from jax.experimental.pallas import tpu as pltpu
from jax.experimental.pallas import tpu_sc as plsc

_N = 16 * 512 * 512
_B = 16384
_NC, _NS, _L = 2, 16, 16
_NW = _NC * _NS
_PER_W = _N // _NW
_CH = 4096
_ROWS_CH = 8
_N_CH = _PER_W // _CH
_GROUPS = _CH // _L


def _mm_body(p_ref, l_ref, mn_ref, mx_ref):
    i = pl.program_id(0)
    lf = l_ref[...].astype(jnp.float32)
    e = 1.0 - p_ref[...] * (2.0 * lf - 1.0)
    mn = jnp.min(e)
    mx = jnp.max(e)

    @pl.when(i == 0)
    def _():
        mn_ref[...] = jnp.full((8, 128), mn, jnp.float32)
        mx_ref[...] = jnp.full((8, 128), mx, jnp.float32)

    @pl.when(i != 0)
    def _():
        mn_ref[...] = jnp.minimum(mn_ref[...], mn)
        mx_ref[...] = jnp.maximum(mx_ref[...], mx)


def _minmax(pred2d, lab2d):
    rows = pred2d.shape[0]
    blk = 1024
    grid = rows // blk
    return pl.pallas_call(
        _mm_body,
        grid=(grid,),
        in_specs=[
            pl.BlockSpec((blk, 512), lambda i: (i, 0)),
            pl.BlockSpec((blk, 512), lambda i: (i, 0)),
        ],
        out_specs=[
            pl.BlockSpec((8, 128), lambda i: (0, 0)),
            pl.BlockSpec((8, 128), lambda i: (0, 0)),
        ],
        out_shape=[
            jax.ShapeDtypeStruct((8, 128), jnp.float32),
            jax.ShapeDtypeStruct((8, 128), jnp.float32),
        ],
    )(pred2d, lab2d)


def _hist_body(pred_hbm, lab_hbm, mn_hbm, mx_hbm, h_out, s_out,
               pb0, pb1, lb0, lb1, mnb, mxb, hvm, svm, sem0, sem1):
    wid = lax.axis_index("s") * _NC + lax.axis_index("c")
    batch = wid // 2
    row0 = (wid % 2) * 256

    @plsc.parallel_loop(0, (2 * _B) // _L, unroll=8)
    def _(j):
        z = jnp.zeros((_L,), jnp.float32)
        hvm[pl.ds(j * _L, _L)] = z
        svm[pl.ds(j * _L, _L)] = z

    pltpu.sync_copy(mn_hbm, mnb)
    pltpu.sync_copy(mx_hbm, mxb)
    mn = mnb[...]
    mx = mxb[...]
    scale = (_B - 1.0) / jnp.maximum(mx - mn, 1e-30)
    a_vec = (mx - 1.0) * scale
    two_scale = scale + scale
    ones = jnp.full((_L,), 1.0, jnp.float32)
    top = jnp.full((_L,), _B - 1.0, jnp.float32)
    zero = jnp.zeros((_L,), jnp.float32)

    def _start(c, pb, lb, sem):
        r = row0 + c * _ROWS_CH
        pltpu.async_copy(pred_hbm.at[batch, pl.ds(r, _ROWS_CH), :], pb, sem)
        pltpu.async_copy(lab_hbm.at[batch, pl.ds(r, _ROWS_CH), :], lb, sem)

    def _wait(c, pb, lb, sem):
        r = row0 + c * _ROWS_CH
        pltpu.make_async_copy(
            pred_hbm.at[batch, pl.ds(r, _ROWS_CH), :], pb, sem).wait()
        pltpu.make_async_copy(
            lab_hbm.at[batch, pl.ds(r, _ROWS_CH), :], lb, sem).wait()

    def _process(pb, lb):
        @plsc.parallel_loop(0, _GROUPS, unroll=8)
        def _(j):
            row = j // 32
            off = (j % 32) * _L
            p = pb[row, pl.ds(off, _L)]
            lab = lb[row, pl.ds(off, _L)]
            lf = lab.astype(jnp.float32)
            t = a_vec + p * (lf * two_scale - scale)
            tc = jnp.minimum(jnp.maximum(t, zero), top)
            idx = tc.astype(jnp.int32) + lab * _B
            plsc.addupdate_scatter(hvm, [idx], ones)
            plsc.addupdate_scatter(svm, [idx], t)

    _start(0, pb0, lb0, sem0)

    def pair_body(cc, carry):
        c0 = 2 * cc
        c1 = c0 + 1
        _start(c1, pb1, lb1, sem1)
        _wait(c0, pb0, lb0, sem0)
        _process(pb0, lb0)
        cn = lax.rem(c0 + 2, _N_CH)
        _start(cn, pb0, lb0, sem0)
        _wait(c1, pb1, lb1, sem1)
        _process(pb1, lb1)
        return carry

    lax.fori_loop(0, _N_CH // 2, pair_body, 0)
    _wait(0, pb0, lb0, sem0)

    pltpu.sync_copy(hvm, h_out.at[wid])
    pltpu.sync_copy(svm, s_out.at[wid])


def _histogram(pred_flat, lab_flat, mnv, mxv):
    mesh = plsc.VectorSubcoreMesh(core_axis_name="c", subcore_axis_name="s")
    return pl.kernel(
        _hist_body,
        mesh=mesh,
        compiler_params=pltpu.CompilerParams(
            needs_layout_passes=False, use_tc_tiling_on_sc=True),
        out_type=[
            jax.ShapeDtypeStruct((_NW, 2 * _B), jnp.float32),
            jax.ShapeDtypeStruct((_NW, 2 * _B), jnp.float32),
        ],
        scratch_types=[
            pltpu.VMEM((_ROWS_CH, 512), jnp.float32),
            pltpu.VMEM((_ROWS_CH, 512), jnp.float32),
            pltpu.VMEM((_ROWS_CH, 512), jnp.int32),
            pltpu.VMEM((_ROWS_CH, 512), jnp.int32),
            pltpu.VMEM((_L,), jnp.float32),
            pltpu.VMEM((_L,), jnp.float32),
            pltpu.VMEM((2 * _B,), jnp.float32),
            pltpu.VMEM((2 * _B,), jnp.float32),
            pltpu.SemaphoreType.DMA,
            pltpu.SemaphoreType.DMA,
        ],
    )(pred_flat, lab_flat, mnv, mxv)


def _excl_prefix(x):
    inc = x
    for k in (1, 2, 4, 8, 16, 32, 64):
        shifted = jnp.concatenate(
            [jnp.zeros((128, k), jnp.float32), inc[:, : 128 - k]], axis=1)
        inc = inc + shifted
    rowtot = jnp.broadcast_to(inc[:, 127:128], (128, 128))
    cumr = rowtot
    for k in (1, 2, 4, 8, 16, 32, 64):
        shifted = jnp.concatenate(
            [jnp.zeros((k, 128), jnp.float32), cumr[: 128 - k, :]], axis=0)
        cumr = cumr + shifted
    return inc - x + (cumr - rowtot)


def _f(e):
    return jnp.where(e > 0.0, e + 1.0, jnp.exp(e))


def _fin_body(h_ref, s_ref, mn_ref, mx_ref, o_ref):
    Hn = jnp.sum(h_ref[:, 0], axis=0)
    Hp = jnp.sum(h_ref[:, 1], axis=0)
    Sn = jnp.sum(s_ref[:, 0], axis=0)
    Sp = jnp.sum(s_ref[:, 1], axis=0)
    G = jnp.sum(Hp)
    mns = jnp.max(mn_ref[...])
    mxs = jnp.max(mx_ref[...])
    scale = (_B - 1.0) / jnp.maximum(mxs - mns, 1e-30)
    ep = mxs - (Sp / jnp.maximum(Hp, 1.0)) / scale
    en = mxs - (Sn / jnp.maximum(Hn, 1.0)) / scale
    Ppos = _excl_prefix(Hp)
    Pneg = _excl_prefix(Hn)
    u0 = G + Pneg
    fp = _f(ep)
    fn = _f(en)
    pos_t = Hp * fp / jnp.maximum(u0, 1.0)
    I = G - Ppos - Hp
    neg_t = Hn * fn * I / jnp.maximum(u0 * (u0 + Hn), 1.0)
    loss = jnp.sum(pos_t) + jnp.sum(neg_t)
    loss = jnp.where(G == 0.0, _f(mxs), loss)
    o_ref[0, 0] = loss


def _finish(h4, s4, mn8, mx8):
    return pl.pallas_call(
        _fin_body,
        in_specs=[
            pl.BlockSpec(memory_space=pltpu.VMEM),
            pl.BlockSpec(memory_space=pltpu.VMEM),
            pl.BlockSpec(memory_space=pltpu.VMEM),
            pl.BlockSpec(memory_space=pltpu.VMEM),
        ],
        out_specs=pl.BlockSpec(memory_space=pltpu.SMEM),
        out_shape=jax.ShapeDtypeStruct((1, 1), jnp.float32),
    )(h4, s4, mn8, mx8)


def kernel(y_pred, y_true):
    lab = y_true.astype(jnp.int32)
    pred2d = y_pred.reshape(8192, 512)
    lab2d = lab.reshape(8192, 512)
    mn8, mx8 = _minmax(pred2d, lab2d)
    mnv = mn8[0, :_L]
    mxv = mx8[0, :_L]
    h, s = _histogram(y_pred, lab, mnv, mxv)
    h4 = h.reshape(_NW, 2, 128, 128)
    s4 = s.reshape(_NW, 2, 128, 128)
    out = _finish(h4, s4, mn8, mx8)
    return out.reshape(())


# speedup vs baseline: 102.2745x; 1.1278x over previous
import functools

import jax
import jax.numpy as jnp
from jax import lax
from jax.experimental import pallas as pl
from jax.experimental.pallas import tpu as pltpu
from jax.experimental.pallas import tpu_sc as plsc

_N = 16 * 512 * 512
_B = 16384
_ROWS = 2 * _B // 128
_NC, _NS, _L = 2, 16, 16
_NW = _NC * _NS
_PER_W = _N // _NW
_CH = 4096
_ROWS_CH = 8
_N_CH = _PER_W // _CH
_GROUPS = _CH // _L


def _prep_body(p_ref, l_ref, w_ref, mn_ref, mx_ref):
    i = pl.program_id(0)
    lab = l_ref[...]
    p = p_ref[...]
    w = p * (2.0 * lab.astype(jnp.float32) - 1.0)
    bits = jax.lax.bitcast_convert_type(w, jnp.int32)
    bits = jnp.bitwise_or(jnp.bitwise_and(bits, jnp.int32(-2)), lab)
    w = jax.lax.bitcast_convert_type(bits, jnp.float32)
    w_ref[...] = w
    mn = jnp.min(w)
    mx = jnp.max(w)

    @pl.when(i == 0)
    def _():
        mn_ref[...] = jnp.full((8, 128), mn, jnp.float32)
        mx_ref[...] = jnp.full((8, 128), mx, jnp.float32)

    @pl.when(i != 0)
    def _():
        mn_ref[...] = jnp.minimum(mn_ref[...], mn)
        mx_ref[...] = jnp.maximum(mx_ref[...], mx)


def _prep(pred2d, lab2d):
    rows = pred2d.shape[0]
    blk = 1024
    grid = rows // blk
    return pl.pallas_call(
        _prep_body,
        grid=(grid,),
        in_specs=[
            pl.BlockSpec((blk, 512), lambda i: (i, 0)),
            pl.BlockSpec((blk, 512), lambda i: (i, 0)),
        ],
        out_specs=[
            pl.BlockSpec((blk, 512), lambda i: (i, 0)),
            pl.BlockSpec((8, 128), lambda i: (0, 0)),
            pl.BlockSpec((8, 128), lambda i: (0, 0)),
        ],
        out_shape=[
            jax.ShapeDtypeStruct((rows, 512), jnp.float32),
            jax.ShapeDtypeStruct((8, 128), jnp.float32),
            jax.ShapeDtypeStruct((8, 128), jnp.float32),
        ],
    )(pred2d, lab2d)


def _hist_body(w_hbm, mn_hbm, mx_hbm, h_out, s_out,
               wb0, wb1, mnb, mxb, hvm, svm, sem0, sem1):
    wid = lax.axis_index("s") * _NC + lax.axis_index("c")
    row0 = wid * (_PER_W // 512)

    @plsc.parallel_loop(0, (2 * _B) // _L, unroll=8)
    def _(j):
        z = jnp.zeros((_L,), jnp.float32)
        r = j // 8
        off = (j % 8) * _L
        hvm[r, pl.ds(off, _L)] = z
        svm[r, pl.ds(off, _L)] = z

    pltpu.sync_copy(mn_hbm, mnb)
    pltpu.sync_copy(mx_hbm, mxb)
    wmn = mnb[...]
    wmx = mxb[...]
    scale = (_B - 1.0) / jnp.maximum(wmx - wmn, 1e-30)
    neg_mn_scale = -wmn * scale
    ones = jnp.full((_L,), 1.0, jnp.float32)
    lsb = jnp.full((_L,), 1, jnp.int32)
    c127 = jnp.full((_L,), 127, jnp.int32)

    def _start(c, wb, sem):
        r = row0 + c * _ROWS_CH
        pltpu.async_copy(w_hbm.at[pl.ds(r, _ROWS_CH), :], wb, sem)

    def _wait(c, wb, sem):
        r = row0 + c * _ROWS_CH
        pltpu.make_async_copy(
            w_hbm.at[pl.ds(r, _ROWS_CH), :], wb, sem).wait()

    def _process(wb):
        @plsc.parallel_loop(0, _GROUPS, unroll=8)
        def _(j):
            row = j // 32
            off = (j % 32) * _L
            w = wb[row, pl.ds(off, _L)]
            labbit = jnp.bitwise_and(plsc.bitcast(w, jnp.int32), lsb)
            t = w * scale + neg_mn_scale
            idx = t.astype(jnp.int32) + jnp.left_shift(labbit, 14)
            hr = jnp.right_shift(idx, 7)
            hc = jnp.bitwise_and(idx, c127)
            plsc.addupdate_scatter(hvm, [hr, hc], ones)
            plsc.addupdate_scatter(svm, [hr, hc], t)

    _start(0, wb0, sem0)

    def pair_body(cc, carry):
        c0 = 2 * cc
        c1 = c0 + 1
        _start(c1, wb1, sem1)
        _wait(c0, wb0, sem0)
        _process(wb0)
        cn = lax.rem(c0 + 2, _N_CH)
        _start(cn, wb0, sem0)
        _wait(c1, wb1, sem1)
        _process(wb1)
        return carry

    lax.fori_loop(0, _N_CH // 2, pair_body, 0)
    _wait(0, wb0, sem0)

    pltpu.sync_copy(hvm, h_out.at[wid])
    pltpu.sync_copy(svm, s_out.at[wid])


def _histogram(w2d, mnv, mxv):
    mesh = plsc.VectorSubcoreMesh(core_axis_name="c", subcore_axis_name="s")
    return pl.kernel(
        _hist_body,
        mesh=mesh,
        compiler_params=pltpu.CompilerParams(
            needs_layout_passes=False, use_tc_tiling_on_sc=True),
        out_type=[
            jax.ShapeDtypeStruct((_NW, _ROWS, 128), jnp.float32),
            jax.ShapeDtypeStruct((_NW, _ROWS, 128), jnp.float32),
        ],
        scratch_types=[
            pltpu.VMEM((_ROWS_CH, 512), jnp.float32),
            pltpu.VMEM((_ROWS_CH, 512), jnp.float32),
            pltpu.VMEM((_L,), jnp.float32),
            pltpu.VMEM((_L,), jnp.float32),
            pltpu.VMEM((_ROWS, 128), jnp.float32),
            pltpu.VMEM((_ROWS, 128), jnp.float32),
            pltpu.SemaphoreType.DMA,
            pltpu.SemaphoreType.DMA,
        ],
    )(w2d, mnv, mxv)


def _excl_prefix(x):
    inc = x
    for k in (1, 2, 4, 8, 16, 32, 64):
        shifted = jnp.concatenate(
            [jnp.zeros((128, k), jnp.float32), inc[:, : 128 - k]], axis=1)
        inc = inc + shifted
    rowtot = jnp.broadcast_to(inc[:, 127:128], (128, 128))
    cumr = rowtot
    for k in (1, 2, 4, 8, 16, 32, 64):
        shifted = jnp.concatenate(
            [jnp.zeros((k, 128), jnp.float32), cumr[: 128 - k, :]], axis=0)
        cumr = cumr + shifted
    return inc - x + (cumr - rowtot)


def _f(e):
    return jnp.where(e > 0.0, e + 1.0, jnp.exp(e))


def _fin_body(h_ref, s_ref, mn_ref, mx_ref, o_ref):
    Hn = jnp.sum(h_ref[:, :128, :], axis=0)
    Hp = jnp.sum(h_ref[:, 128:, :], axis=0)
    Sn = jnp.sum(s_ref[:, :128, :], axis=0)
    Sp = jnp.sum(s_ref[:, 128:, :], axis=0)
    G = jnp.sum(Hp)
    wmn = jnp.max(mn_ref[...])
    wmx = jnp.max(mx_ref[...])
    emx = 1.0 - wmn
    scale = (_B - 1.0) / jnp.maximum(wmx - wmn, 1e-30)
    ep = emx - (Sp / jnp.maximum(Hp, 1.0)) / scale
    en = emx - (Sn / jnp.maximum(Hn, 1.0)) / scale
    Ppos = _excl_prefix(Hp)
    Pneg = _excl_prefix(Hn)
    u0 = G + Pneg
    fp = _f(ep)
    fn = _f(en)
    pos_t = Hp * fp / jnp.maximum(u0, 1.0)
    I = G - Ppos - Hp
    neg_t = Hn * fn * I / jnp.maximum(u0 * (u0 + Hn), 1.0)
    loss = jnp.sum(pos_t) + jnp.sum(neg_t)
    loss = jnp.where(G == 0.0, _f(emx), loss)
    o_ref[0, 0] = loss


def _finish(h3, s3, mn8, mx8):
    return pl.pallas_call(
        _fin_body,
        in_specs=[
            pl.BlockSpec(memory_space=pltpu.VMEM),
            pl.BlockSpec(memory_space=pltpu.VMEM),
            pl.BlockSpec(memory_space=pltpu.VMEM),
            pl.BlockSpec(memory_space=pltpu.VMEM),
        ],
        out_specs=pl.BlockSpec(memory_space=pltpu.SMEM),
        out_shape=jax.ShapeDtypeStruct((1, 1), jnp.float32),
    )(h3, s3, mn8, mx8)


def kernel(y_pred, y_true):
    lab = y_true.astype(jnp.int32)
    pred2d = y_pred.reshape(8192, 512)
    lab2d = lab.reshape(8192, 512)
    w2d, mn8, mx8 = _prep(pred2d, lab2d)
    mnv = mn8[0, :_L]
    mxv = mx8[0, :_L]
    h, s = _histogram(w2d, mnv, mxv)
    out = _finish(h, s, mn8, mx8)
    return out.reshape(())


# speedup vs baseline: 109.6188x; 1.0718x over previous
import functools

import jax
import jax.numpy as jnp
from jax import lax
from jax.experimental import pallas as pl
from jax.experimental.pallas import tpu as pltpu
from jax.experimental.pallas import tpu_sc as plsc

_N = 16 * 512 * 512
_B = 4096
_LOGB = 12
_ROWS = 2 * _B // 128
_CROWS = _B // 128
_NC, _NS, _L = 2, 16, 16
_NW = _NC * _NS
_PER_W = _N // _NW
_CH = 4096
_ROWS_CH = 8
_N_CH = _PER_W // _CH
_GROUPS = _CH // _L


def _prep_body(p_ref, l_ref, w_ref, mn_ref, mx_ref):
    i = pl.program_id(0)
    lab = l_ref[...]
    p = p_ref[...]
    w = jnp.where(lab == 1, p, -p)
    bits = jax.lax.bitcast_convert_type(w, jnp.int32)
    bits = jnp.bitwise_or(jnp.bitwise_and(bits, jnp.int32(-2)), lab)
    w = jax.lax.bitcast_convert_type(bits, jnp.float32)
    w_ref[...] = w
    mn = jnp.min(w)
    mx = jnp.max(w)

    @pl.when(i == 0)
    def _():
        mn_ref[...] = jnp.full((8, 128), mn, jnp.float32)
        mx_ref[...] = jnp.full((8, 128), mx, jnp.float32)

    @pl.when(i != 0)
    def _():
        mn_ref[...] = jnp.minimum(mn_ref[...], mn)
        mx_ref[...] = jnp.maximum(mx_ref[...], mx)


def _prep(pred2d, lab2d):
    rows = pred2d.shape[0]
    blk = 1024
    grid = rows // blk
    return pl.pallas_call(
        _prep_body,
        grid=(grid,),
        in_specs=[
            pl.BlockSpec((blk, 512), lambda i: (i, 0)),
            pl.BlockSpec((blk, 512), lambda i: (i, 0)),
        ],
        out_specs=[
            pl.BlockSpec((blk, 512), lambda i: (i, 0)),
            pl.BlockSpec((8, 128), lambda i: (0, 0)),
            pl.BlockSpec((8, 128), lambda i: (0, 0)),
        ],
        out_shape=[
            jax.ShapeDtypeStruct((rows, 512), jnp.float32),
            jax.ShapeDtypeStruct((8, 128), jnp.float32),
            jax.ShapeDtypeStruct((8, 128), jnp.float32),
        ],
    )(pred2d, lab2d)


def _hist_body(w_hbm, mn_hbm, mx_hbm, h_out, s_out,
               wb0, wb1, mnb, mxb, hvm, svm, sem0, sem1):
    wid = lax.axis_index("s") * _NC + lax.axis_index("c")
    row0 = wid * (_PER_W // 512)

    @plsc.parallel_loop(0, (2 * _B) // _L, unroll=8)
    def _(j):
        z = jnp.zeros((_L,), jnp.float32)
        r = j // 8
        off = (j % 8) * _L
        hvm[r, pl.ds(off, _L)] = z
        svm[r, pl.ds(off, _L)] = z

    pltpu.sync_copy(mn_hbm, mnb)
    pltpu.sync_copy(mx_hbm, mxb)
    wmn = mnb[...]
    wmx = mxb[...]
    scale = (_B - 1.0) / jnp.maximum(wmx - wmn, 1e-30)
    neg_mn_scale = -wmn * scale
    ones = jnp.full((_L,), 1.0, jnp.float32)
    lsb = jnp.full((_L,), 1, jnp.int32)
    c127 = jnp.full((_L,), 127, jnp.int32)

    def _start(c, wb, sem):
        r = row0 + c * _ROWS_CH
        pltpu.async_copy(w_hbm.at[pl.ds(r, _ROWS_CH), :], wb, sem)

    def _wait(c, wb, sem):
        r = row0 + c * _ROWS_CH
        pltpu.make_async_copy(
            w_hbm.at[pl.ds(r, _ROWS_CH), :], wb, sem).wait()

    def _process(wb):
        @plsc.parallel_loop(0, _GROUPS, unroll=16)
        def _(j):
            row = j // 32
            off = (j % 32) * _L
            w = wb[row, pl.ds(off, _L)]
            labbit = jnp.bitwise_and(plsc.bitcast(w, jnp.int32), lsb)
            t = w * scale + neg_mn_scale
            idx = t.astype(jnp.int32) + jnp.left_shift(labbit, _LOGB)
            hr = jnp.right_shift(idx, 7)
            hc = jnp.bitwise_and(idx, c127)
            plsc.addupdate_scatter(hvm, [hr, hc], ones)
            plsc.addupdate_scatter(svm, [hr, hc], t)

    _start(0, wb0, sem0)

    def pair_body(cc, carry):
        c0 = 2 * cc
        c1 = c0 + 1
        _start(c1, wb1, sem1)
        _wait(c0, wb0, sem0)
        _process(wb0)
        cn = lax.rem(c0 + 2, _N_CH)
        _start(cn, wb0, sem0)
        _wait(c1, wb1, sem1)
        _process(wb1)
        return carry

    lax.fori_loop(0, _N_CH // 2, pair_body, 0)
    _wait(0, wb0, sem0)

    pltpu.sync_copy(hvm, h_out.at[wid])
    pltpu.sync_copy(svm, s_out.at[wid])


def _histogram(w2d, mnv, mxv):
    mesh = plsc.VectorSubcoreMesh(core_axis_name="c", subcore_axis_name="s")
    return pl.kernel(
        _hist_body,
        mesh=mesh,
        compiler_params=pltpu.CompilerParams(
            needs_layout_passes=False, use_tc_tiling_on_sc=True),
        out_type=[
            jax.ShapeDtypeStruct((_NW, _ROWS, 128), jnp.float32),
            jax.ShapeDtypeStruct((_NW, _ROWS, 128), jnp.float32),
        ],
        scratch_types=[
            pltpu.VMEM((_ROWS_CH, 512), jnp.float32),
            pltpu.VMEM((_ROWS_CH, 512), jnp.float32),
            pltpu.VMEM((_L,), jnp.float32),
            pltpu.VMEM((_L,), jnp.float32),
            pltpu.VMEM((_ROWS, 128), jnp.float32),
            pltpu.VMEM((_ROWS, 128), jnp.float32),
            pltpu.SemaphoreType.DMA,
            pltpu.SemaphoreType.DMA,
        ],
    )(w2d, mnv, mxv)


def _excl_prefix(x):
    R = x.shape[0]
    inc = x
    k = 1
    while k < 128:
        shifted = jnp.concatenate(
            [jnp.zeros((R, k), jnp.float32), inc[:, : 128 - k]], axis=1)
        inc = inc + shifted
        k *= 2
    rowtot = jnp.broadcast_to(inc[:, 127:128], (R, 128))
    cumr = rowtot
    k = 1
    while k < R:
        shifted = jnp.concatenate(
            [jnp.zeros((k, 128), jnp.float32), cumr[: R - k, :]], axis=0)
        cumr = cumr + shifted
        k *= 2
    return inc - x + (cumr - rowtot)


def _f(e):
    return jnp.where(e > 0.0, e + 1.0, jnp.exp(e))


def _fin_body(h_ref, s_ref, mn_ref, mx_ref, o_ref):
    Hn = jnp.sum(h_ref[:, :_CROWS, :], axis=0)
    Hp = jnp.sum(h_ref[:, _CROWS:, :], axis=0)
    Sn = jnp.sum(s_ref[:, :_CROWS, :], axis=0)
    Sp = jnp.sum(s_ref[:, _CROWS:, :], axis=0)
    G = jnp.sum(Hp)
    wmn = jnp.max(mn_ref[...])
    wmx = jnp.max(mx_ref[...])
    emx = 1.0 - wmn
    scale = (_B - 1.0) / jnp.maximum(wmx - wmn, 1e-30)
    ep = emx - (Sp / jnp.maximum(Hp, 1.0)) / scale
    en = emx - (Sn / jnp.maximum(Hn, 1.0)) / scale
    Ppos = _excl_prefix(Hp)
    Pneg = _excl_prefix(Hn)
    u0 = G + Pneg
    fp = _f(ep)
    fn = _f(en)
    pos_t = Hp * fp / jnp.maximum(u0, 1.0)
    I = G - Ppos - Hp
    neg_t = Hn * fn * I / jnp.maximum(u0 * (u0 + Hn), 1.0)
    loss = jnp.sum(pos_t) + jnp.sum(neg_t)
    loss = jnp.where(G == 0.0, _f(emx), loss)
    o_ref[0, 0] = loss


def _finish(h3, s3, mn8, mx8):
    return pl.pallas_call(
        _fin_body,
        in_specs=[
            pl.BlockSpec(memory_space=pltpu.VMEM),
            pl.BlockSpec(memory_space=pltpu.VMEM),
            pl.BlockSpec(memory_space=pltpu.VMEM),
            pl.BlockSpec(memory_space=pltpu.VMEM),
        ],
        out_specs=pl.BlockSpec(memory_space=pltpu.SMEM),
        out_shape=jax.ShapeDtypeStruct((1, 1), jnp.float32),
    )(h3, s3, mn8, mx8)


def kernel(y_pred, y_true):
    lab = y_true.astype(jnp.int32)
    pred2d = y_pred.reshape(8192, 512)
    lab2d = lab.reshape(8192, 512)
    w2d, mn8, mx8 = _prep(pred2d, lab2d)
    mnv = mn8[0, :_L]
    mxv = mx8[0, :_L]
    h, s = _histogram(w2d, mnv, mxv)
    out = _finish(h, s, mn8, mx8)
    return out.reshape(())


# speedup vs baseline: 117.8735x; 1.0753x over previous
import functools

import jax
import jax.numpy as jnp
from jax import lax
from jax.experimental import pallas as pl
from jax.experimental.pallas import tpu as pltpu
from jax.experimental.pallas import tpu_sc as plsc

_N = 16 * 512 * 512
_B = 4096
_LOGB = 12
_ROWS = 2 * _B // 128
_CROWS = _B // 128
_NC, _NS, _L = 2, 16, 16
_NW = _NC * _NS
_PER_W = _N // _NW
_CH = 4096
_ROWS_CH = 8
_N_CH = _PER_W // _CH
_GROUPS = _CH // _L


def _mm_body(p_ref, mx_ref):
    i = pl.program_id(0)
    mx = jnp.max(jnp.abs(p_ref[...]))

    @pl.when(i == 0)
    def _():
        mx_ref[...] = jnp.full((8, 128), mx, jnp.float32)

    @pl.when(i != 0)
    def _():
        mx_ref[...] = jnp.maximum(mx_ref[...], mx)


def _maxabs(pred2d):
    rows = pred2d.shape[0]
    blk = 1024
    grid = rows // blk
    return pl.pallas_call(
        _mm_body,
        grid=(grid,),
        in_specs=[pl.BlockSpec((blk, 512), lambda i: (i, 0))],
        out_specs=pl.BlockSpec((8, 128), lambda i: (0, 0)),
        out_shape=jax.ShapeDtypeStruct((8, 128), jnp.float32),
    )(pred2d)


def _hist_body(pred_hbm, lab_hbm, m_hbm, h_out, s_out,
               pb0, pb1, lb0, lb1, mb, hvm, svm, sem0, sem1):
    wid = lax.axis_index("s") * _NC + lax.axis_index("c")
    batch = wid // 2
    row0 = (wid % 2) * 256

    @plsc.parallel_loop(0, (2 * _B) // _L, unroll=8)
    def _(j):
        z = jnp.zeros((_L,), jnp.float32)
        r = j // 8
        off = (j % 8) * _L
        hvm[r, pl.ds(off, _L)] = z
        svm[r, pl.ds(off, _L)] = z

    pltpu.sync_copy(m_hbm, mb)
    m = mb[...]
    scale = (_B - 1.0) / jnp.maximum(m + m, 1e-30)
    mscale = m * scale
    nscale = -scale
    ones = jnp.full((_L,), 1.0, jnp.float32)
    c127 = jnp.full((_L,), 127, jnp.int32)

    def _start(c, pb, lb, sem):
        r = row0 + c * _ROWS_CH
        pltpu.async_copy(pred_hbm.at[batch, pl.ds(r, _ROWS_CH), :], pb, sem)
        pltpu.async_copy(lab_hbm.at[batch, pl.ds(r, _ROWS_CH), :], lb, sem)

    def _wait(c, pb, lb, sem):
        r = row0 + c * _ROWS_CH
        pltpu.make_async_copy(
            pred_hbm.at[batch, pl.ds(r, _ROWS_CH), :], pb, sem).wait()
        pltpu.make_async_copy(
            lab_hbm.at[batch, pl.ds(r, _ROWS_CH), :], lb, sem).wait()

    def _process(pb, lb):
        @plsc.parallel_loop(0, _GROUPS, unroll=16)
        def _(j):
            row = j // 32
            off = (j % 32) * _L
            p = pb[row, pl.ds(off, _L)]
            lab = lb[row, pl.ds(off, _L)]
            sscale = jnp.where(lab == 0, nscale, scale)
            t = p * sscale + mscale
            idx = t.astype(jnp.int32) + jnp.left_shift(lab, _LOGB)
            hr = jnp.right_shift(idx, 7)
            hc = jnp.bitwise_and(idx, c127)
            plsc.addupdate_scatter(hvm, [hr, hc], ones)
            plsc.addupdate_scatter(svm, [hr, hc], t)

    _start(0, pb0, lb0, sem0)

    def pair_body(cc, carry):
        c0 = 2 * cc
        c1 = c0 + 1
        _start(c1, pb1, lb1, sem1)
        _wait(c0, pb0, lb0, sem0)
        _process(pb0, lb0)
        cn = lax.rem(c0 + 2, _N_CH)
        _start(cn, pb0, lb0, sem0)
        _wait(c1, pb1, lb1, sem1)
        _process(pb1, lb1)
        return carry

    lax.fori_loop(0, _N_CH // 2, pair_body, 0)
    _wait(0, pb0, lb0, sem0)

    pltpu.sync_copy(hvm, h_out.at[wid])
    pltpu.sync_copy(svm, s_out.at[wid])


def _histogram(pred3, lab3, mv):
    mesh = plsc.VectorSubcoreMesh(core_axis_name="c", subcore_axis_name="s")
    return pl.kernel(
        _hist_body,
        mesh=mesh,
        compiler_params=pltpu.CompilerParams(
            needs_layout_passes=False, use_tc_tiling_on_sc=True),
        out_type=[
            jax.ShapeDtypeStruct((_NW, _ROWS, 128), jnp.float32),
            jax.ShapeDtypeStruct((_NW, _ROWS, 128), jnp.float32),
        ],
        scratch_types=[
            pltpu.VMEM((_ROWS_CH, 512), jnp.float32),
            pltpu.VMEM((_ROWS_CH, 512), jnp.float32),
            pltpu.VMEM((_ROWS_CH, 512), jnp.int32),
            pltpu.VMEM((_ROWS_CH, 512), jnp.int32),
            pltpu.VMEM((_L,), jnp.float32),
            pltpu.VMEM((_ROWS, 128), jnp.float32),
            pltpu.VMEM((_ROWS, 128), jnp.float32),
            pltpu.SemaphoreType.DMA,
            pltpu.SemaphoreType.DMA,
        ],
    )(pred3, lab3, mv)


def _excl_prefix(x):
    R = x.shape[0]
    inc = x
    k = 1
    while k < 128:
        shifted = jnp.concatenate(
            [jnp.zeros((R, k), jnp.float32), inc[:, : 128 - k]], axis=1)
        inc = inc + shifted
        k *= 2
    rowtot = jnp.broadcast_to(inc[:, 127:128], (R, 128))
    cumr = rowtot
    k = 1
    while k < R:
        shifted = jnp.concatenate(
            [jnp.zeros((k, 128), jnp.float32), cumr[: R - k, :]], axis=0)
        cumr = cumr + shifted
        k *= 2
    return inc - x + (cumr - rowtot)


def _f(e):
    return jnp.where(e > 0.0, e + 1.0, jnp.exp(e))


def _fin_body(h_ref, s_ref, m_ref, o_ref):
    Hn = jnp.sum(h_ref[:, :_CROWS, :], axis=0)
    Hp = jnp.sum(h_ref[:, _CROWS:, :], axis=0)
    Sn = jnp.sum(s_ref[:, :_CROWS, :], axis=0)
    Sp = jnp.sum(s_ref[:, _CROWS:, :], axis=0)
    G = jnp.sum(Hp)
    M = jnp.max(m_ref[...])
    emx = 1.0 + M
    scale = (_B - 1.0) / jnp.maximum(M + M, 1e-30)
    ep = emx - (Sp / jnp.maximum(Hp, 1.0)) / scale
    en = emx - (Sn / jnp.maximum(Hn, 1.0)) / scale
    Ppos = _excl_prefix(Hp)
    Pneg = _excl_prefix(Hn)
    u0 = G + Pneg
    fp = _f(ep)
    fn = _f(en)
    pos_t = Hp * fp / jnp.maximum(u0, 1.0)
    I = G - Ppos - Hp
    neg_t = Hn * fn * I / jnp.maximum(u0 * (u0 + Hn), 1.0)
    loss = jnp.sum(pos_t) + jnp.sum(neg_t)
    bidx = (jax.lax.broadcasted_iota(jnp.int32, (_CROWS, 128), 0) * 128
            + jax.lax.broadcasted_iota(jnp.int32, (_CROWS, 128), 1))
    btop = jnp.min(jnp.where(Hn > 0.0, bidx, jnp.int32(2 ** 30)))
    en_top = jnp.sum(jnp.where(bidx == btop, en, 0.0))
    loss = jnp.where(G == 0.0, _f(en_top), loss)
    o_ref[0, 0] = loss


def _finish(h3, s3, m8):
    return pl.pallas_call(
        _fin_body,
        in_specs=[
            pl.BlockSpec(memory_space=pltpu.VMEM),
            pl.BlockSpec(memory_space=pltpu.VMEM),
            pl.BlockSpec(memory_space=pltpu.VMEM),
        ],
        out_specs=pl.BlockSpec(memory_space=pltpu.SMEM),
        out_shape=jax.ShapeDtypeStruct((1, 1), jnp.float32),
    )(h3, s3, m8)


def kernel(y_pred, y_true):
    lab = y_true.astype(jnp.int32)
    m8 = _maxabs(y_pred.reshape(8192, 512))
    mv = m8[0, :_L]
    h, s = _histogram(y_pred, lab, mv)
    out = _finish(h, s, m8)
    return out.reshape(())
